# Initial kernel scaffold; baseline (speedup 1.0000x reference)
import jax
import jax.numpy as jnp
from jax.experimental import pallas as pl


def kernel(x, edge_index, W1, b1, W2, b2, W3, b3):
    raise NotImplementedError("write your pallas kernel here")


# speedup vs baseline: 15.2530x; 15.2530x over previous
import jax
import jax.numpy as jnp
from jax import lax
from jax.experimental import pallas as pl
from jax.experimental.pallas import tpu as pltpu
from jax.experimental.pallas import tpu_sc as plsc

N = 10000
NP = 10112
D = 128
E = 320000
NC = 2
NS = 16
NW = NC * NS
CH = 128
E_T = 10240
NCH = E_T // CH
BLK = 8
ROWS_T = NP // NS

_mesh = plsc.VectorSubcoreMesh(core_axis_name="c", subcore_axis_name="s",
                               num_cores=NC, num_subcores=NS)


def _zero_vec():
    return jnp.zeros((16,), jnp.float32)


def _agg_body(u_hbm, src32, dst32, out_hbm, acc, src_v, dst_v, rows0, sem0):
    c = lax.axis_index("c")
    s = lax.axis_index("s")
    w = c * NS + s
    pltpu.sync_copy(src32.at[w], src_v)
    pltpu.sync_copy(dst32.at[w], dst_v)
    for i in range(BLK):
        for j in range(D // 16):
            rows0[i, pl.ds(j * 16, 16)] = _zero_vec()

    def zero_step(i, carry):
        pltpu.sync_copy(rows0.at[pl.ds(0, BLK)],
                        acc.at[pl.ds(s * ROWS_T + i * BLK, BLK)])
        return carry

    lax.fori_loop(0, ROWS_T // BLK, zero_step, 0)
    plsc.subcore_barrier()

    def win(j, carry):
        pltpu.async_copy(u_hbm.at[src_v.at[j]], rows0, sem0).wait()
        pltpu.sync_copy(rows0, acc.at[dst_v.at[j]], add=True)
        return carry

    lax.fori_loop(0, NCH, win, 0)
    plsc.subcore_barrier()
    pltpu.sync_copy(acc.at[pl.ds(s * ROWS_T, ROWS_T)],
                    out_hbm.at[c, pl.ds(s * ROWS_T, ROWS_T)])


_agg_kernel = pl.kernel(
    _agg_body,
    out_type=jax.ShapeDtypeStruct((NC, NP, D), jnp.float32),
    mesh=_mesh,
    scratch_types=[
        pltpu.VMEM_SHARED((NP, D), jnp.float32),
        pltpu.VMEM((NCH, CH), jnp.int32),
        pltpu.VMEM((NCH, CH), jnp.int32),
        pltpu.VMEM((CH, D), jnp.float32),
        pltpu.SemaphoreType.DMA,
    ],
)


def _tc_first_body(deg_ref, x_ref, w_ref, dis_ref, u_ref):
    deg = deg_ref[0, :, 0:1] + deg_ref[1, :, 0:1] + 1.0
    dis = lax.rsqrt(deg)
    dis_ref[...] = dis
    h = jnp.dot(x_ref[...], w_ref[...], preferred_element_type=jnp.float32)
    u_ref[...] = dis * h


_tc_first = pl.pallas_call(
    _tc_first_body,
    out_shape=(
        jax.ShapeDtypeStruct((NP, 1), jnp.float32),
        jax.ShapeDtypeStruct((NP, D), jnp.float32),
    ),
)


def _tc_mid_body(sp_ref, u_ref, dis_ref, b_ref, w_ref, un_ref):
    dis = dis_ref[...]
    agg = sp_ref[0] + sp_ref[1] + u_ref[...]
    a = jnp.maximum(dis * agg + b_ref[...], 0.0)
    un_ref[...] = dis * jnp.dot(a, w_ref[...],
                                preferred_element_type=jnp.float32)


_tc_mid = pl.pallas_call(
    _tc_mid_body,
    out_shape=jax.ShapeDtypeStruct((NP, D), jnp.float32),
)


def _tc_last_body(sp_ref, u_ref, dis_ref, b_ref, out_ref):
    agg = sp_ref[0] + sp_ref[1] + u_ref[...]
    out_ref[...] = jnp.maximum(dis_ref[...] * agg + b_ref[...], 0.0)


_tc_last = pl.pallas_call(
    _tc_last_body,
    out_shape=jax.ShapeDtypeStruct((NP, D), jnp.float32),
)


def _pad_edges(idx):
    per_tile = idx.reshape(NW, E // NW)
    pad = N + (jnp.arange(E_T - E // NW, dtype=jnp.int32) % (NP - N))
    pad = jnp.broadcast_to(pad, (NW, E_T - E // NW))
    return jnp.concatenate([per_tile, pad], axis=1).reshape(NW, NCH, CH)


def kernel(x, edge_index, W1, b1, W2, b2, W3, b3):
    src32 = _pad_edges(edge_index[0].astype(jnp.int32))
    dst32 = _pad_edges(edge_index[1].astype(jnp.int32))
    xp = jnp.zeros((NP, D), jnp.float32).at[:N].set(x)
    ones = jnp.zeros((NP, D), jnp.float32).at[:N].set(1.0)
    b1 = b1.reshape(1, D)
    b2 = b2.reshape(1, D)
    b3 = b3.reshape(1, D)

    deg_p = _agg_kernel(ones, src32, dst32)
    dis, u1 = _tc_first(deg_p, xp, W1)
    s1 = _agg_kernel(u1, src32, dst32)
    u2 = _tc_mid(s1, u1, dis, b1, W2)
    s2 = _agg_kernel(u2, src32, dst32)
    u3 = _tc_mid(s2, u2, dis, b2, W3)
    s3 = _agg_kernel(u3, src32, dst32)
    out = _tc_last(s3, u3, dis, b3)
    return out[:N]


# speedup vs baseline: 22.8797x; 1.5000x over previous
import jax
import jax.numpy as jnp
from jax import lax
from jax.experimental import pallas as pl
from jax.experimental.pallas import tpu as pltpu
from jax.experimental.pallas import tpu_sc as plsc

N = 10000
NP = 10112
D = 128
E = 320000
NC = 2
NS = 16
NW = NC * NS
CH = 128
E_T = 10240
NCH = E_T // CH
BLK = 8
ROWS_T = NP // NS

_mesh = plsc.VectorSubcoreMesh(core_axis_name="c", subcore_axis_name="s",
                               num_cores=NC, num_subcores=NS)


def _zero_vec():
    return jnp.zeros((16,), jnp.float32)


DBLK = 16
NBLK = NCH // DBLK


def _agg_body(u_hbm, src32, dst32, out_hbm, acc, src_v, dstb,
              rows0, rows1, sem0, sem1):
    c = lax.axis_index("c")
    s = lax.axis_index("s")
    w = c * NS + s
    pltpu.sync_copy(src32.at[w], src_v)
    for i in range(BLK):
        for j in range(D // 16):
            rows0[i, pl.ds(j * 16, 16)] = _zero_vec()

    def zero_step(i, carry):
        pltpu.sync_copy(rows0.at[pl.ds(0, BLK)],
                        acc.at[pl.ds(s * ROWS_T + i * BLK, BLK)])
        return carry

    lax.fori_loop(0, ROWS_T // BLK, zero_step, 0)
    plsc.subcore_barrier()

    pltpu.async_copy(u_hbm.at[src_v.at[0]], rows0, sem0)

    def block(k, carry):
        pltpu.sync_copy(dst32.at[w, pl.ds(k * DBLK, DBLK)], dstb)

        def pair(i, carry2):
            g = k * DBLK + 2 * i
            pltpu.async_copy(u_hbm.at[src_v.at[g + 1]], rows1, sem1)
            pltpu.make_async_copy(u_hbm.at[src_v.at[g]], rows0, sem0).wait()
            pltpu.sync_copy(rows0, acc.at[dstb.at[2 * i]], add=True)
            pltpu.async_copy(u_hbm.at[src_v.at[g + 2]], rows0, sem0)
            pltpu.make_async_copy(u_hbm.at[src_v.at[g + 1]], rows1, sem1).wait()
            pltpu.sync_copy(rows1, acc.at[dstb.at[2 * i + 1]], add=True)
            return carry2

        npairs = DBLK // 2
        lax.fori_loop(0, npairs, pair, 0)
        return carry

    lax.fori_loop(0, NBLK - 1, block, 0)
    k = NBLK - 1
    pltpu.sync_copy(dst32.at[w, pl.ds(k * DBLK, DBLK)], dstb)

    def pair_last(i, carry2):
        g = k * DBLK + 2 * i
        pltpu.async_copy(u_hbm.at[src_v.at[g + 1]], rows1, sem1)
        pltpu.make_async_copy(u_hbm.at[src_v.at[g]], rows0, sem0).wait()
        pltpu.sync_copy(rows0, acc.at[dstb.at[2 * i]], add=True)
        pltpu.async_copy(u_hbm.at[src_v.at[g + 2]], rows0, sem0)
        pltpu.make_async_copy(u_hbm.at[src_v.at[g + 1]], rows1, sem1).wait()
        pltpu.sync_copy(rows1, acc.at[dstb.at[2 * i + 1]], add=True)
        return carry2

    lax.fori_loop(0, DBLK // 2 - 1, pair_last, 0)
    g = NCH - 2
    pltpu.async_copy(u_hbm.at[src_v.at[g + 1]], rows1, sem1)
    pltpu.make_async_copy(u_hbm.at[src_v.at[g]], rows0, sem0).wait()
    pltpu.sync_copy(rows0, acc.at[dstb.at[DBLK - 2]], add=True)
    pltpu.make_async_copy(u_hbm.at[src_v.at[g + 1]], rows1, sem1).wait()
    pltpu.sync_copy(rows1, acc.at[dstb.at[DBLK - 1]], add=True)

    plsc.subcore_barrier()
    pltpu.sync_copy(acc.at[pl.ds(s * ROWS_T, ROWS_T)],
                    out_hbm.at[c, pl.ds(s * ROWS_T, ROWS_T)])


_agg_kernel = pl.kernel(
    _agg_body,
    out_type=jax.ShapeDtypeStruct((NC, NP, D), jnp.float32),
    mesh=_mesh,
    scratch_types=[
        pltpu.VMEM_SHARED((NP, D), jnp.float32),
        pltpu.VMEM((NCH, CH), jnp.int32),
        pltpu.VMEM((DBLK, CH), jnp.int32),
        pltpu.VMEM((CH, D), jnp.float32),
        pltpu.VMEM((CH, D), jnp.float32),
        pltpu.SemaphoreType.DMA,
        pltpu.SemaphoreType.DMA,
    ],
)


def _tc_first_body(deg_ref, x_ref, w_ref, dis_ref, u_ref):
    deg = deg_ref[0, :, 0:1] + deg_ref[1, :, 0:1] + 1.0
    dis = lax.rsqrt(deg)
    dis_ref[...] = dis
    h = jnp.dot(x_ref[...], w_ref[...], preferred_element_type=jnp.float32)
    u_ref[...] = dis * h


_tc_first = pl.pallas_call(
    _tc_first_body,
    out_shape=(
        jax.ShapeDtypeStruct((NP, 1), jnp.float32),
        jax.ShapeDtypeStruct((NP, D), jnp.float32),
    ),
)


def _tc_mid_body(sp_ref, u_ref, dis_ref, b_ref, w_ref, un_ref):
    dis = dis_ref[...]
    agg = sp_ref[0] + sp_ref[1] + u_ref[...]
    a = jnp.maximum(dis * agg + b_ref[...], 0.0)
    un_ref[...] = dis * jnp.dot(a, w_ref[...],
                                preferred_element_type=jnp.float32)


_tc_mid = pl.pallas_call(
    _tc_mid_body,
    out_shape=jax.ShapeDtypeStruct((NP, D), jnp.float32),
)


def _tc_last_body(sp_ref, u_ref, dis_ref, b_ref, out_ref):
    agg = sp_ref[0] + sp_ref[1] + u_ref[...]
    out_ref[...] = jnp.maximum(dis_ref[...] * agg + b_ref[...], 0.0)


_tc_last = pl.pallas_call(
    _tc_last_body,
    out_shape=jax.ShapeDtypeStruct((NP, D), jnp.float32),
)


def _pad_edges(idx):
    per_tile = idx.reshape(NW, E // NW)
    pad = N + (jnp.arange(E_T - E // NW, dtype=jnp.int32) % (NP - N))
    pad = jnp.broadcast_to(pad, (NW, E_T - E // NW))
    return jnp.concatenate([per_tile, pad], axis=1).reshape(NW, NCH, CH)


def kernel(x, edge_index, W1, b1, W2, b2, W3, b3):
    src32 = _pad_edges(edge_index[0].astype(jnp.int32))
    dst32 = _pad_edges(edge_index[1].astype(jnp.int32))
    xp = jnp.zeros((NP, D), jnp.float32).at[:N].set(x)
    ones = jnp.zeros((NP, D), jnp.float32).at[:N].set(1.0)
    b1 = b1.reshape(1, D)
    b2 = b2.reshape(1, D)
    b3 = b3.reshape(1, D)

    deg_p = _agg_kernel(ones, src32, dst32)
    dis, u1 = _tc_first(deg_p, xp, W1)
    s1 = _agg_kernel(u1, src32, dst32)
    u2 = _tc_mid(s1, u1, dis, b1, W2)
    s2 = _agg_kernel(u2, src32, dst32)
    u3 = _tc_mid(s2, u2, dis, b2, W3)
    s3 = _agg_kernel(u3, src32, dst32)
    out = _tc_last(s3, u3, dis, b3)
    return out[:N]


# speedup vs baseline: 24.9486x; 1.0904x over previous
import jax
import jax.numpy as jnp
from jax import lax
from jax.experimental import pallas as pl
from jax.experimental.pallas import tpu as pltpu
from jax.experimental.pallas import tpu_sc as plsc

N = 10000
NP = 10112
D = 128
E = 320000
NC = 2
NS = 16
NW = NC * NS
CH = 128
E_T = 10240
NCH = E_T // CH
BLK = 8
ROWS_T = NP // NS

_mesh = plsc.VectorSubcoreMesh(core_axis_name="c", subcore_axis_name="s",
                               num_cores=NC, num_subcores=NS)


def _zero_vec():
    return jnp.zeros((16,), jnp.float32)


def _deg_body(dst32, deg_out, acc, dst_v, ones_v, zbuf):
    c = lax.axis_index("c")
    s = lax.axis_index("s")
    w = c * NS + s
    pltpu.sync_copy(dst32.at[w], dst_v)
    for i in range(BLK):
        for j in range(D // 16):
            zbuf[i, pl.ds(j * 16, 16)] = _zero_vec()
    for i in range(CH):
        for j in range(D // 16):
            ones_v[i, pl.ds(j * 16, 16)] = jnp.ones((16,), jnp.float32)

    def zero_step(i, carry):
        pltpu.sync_copy(zbuf, acc.at[pl.ds(s * ROWS_T + i * BLK, BLK)])
        return carry

    lax.fori_loop(0, ROWS_T // BLK, zero_step, 0)
    plsc.subcore_barrier()

    def chunk(g, carry):
        pltpu.sync_copy(ones_v, acc.at[dst_v.at[g]], add=True)
        return carry

    lax.fori_loop(0, NCH, chunk, 0)
    plsc.subcore_barrier()
    pltpu.sync_copy(acc.at[pl.ds(s * ROWS_T, ROWS_T)],
                    deg_out.at[c, pl.ds(s * ROWS_T, ROWS_T)])


_deg_kernel = pl.kernel(
    _deg_body,
    out_type=jax.ShapeDtypeStruct((NC, NP, D), jnp.float32),
    mesh=_mesh,
    scratch_types=[
        pltpu.VMEM_SHARED((NP, D), jnp.float32),
        pltpu.VMEM((NCH, CH), jnp.int32),
        pltpu.VMEM((CH, D), jnp.float32),
        pltpu.VMEM((BLK, D), jnp.float32),
    ],
)


DBLK = 16
NBLK = NCH // DBLK


def _agg_body(u_hbm, src32, dst32, out_hbm, acc, src_v, dstb,
              rows0, rows1, sem0, sem1):
    c = lax.axis_index("c")
    s = lax.axis_index("s")
    w = c * NS + s
    pltpu.sync_copy(src32.at[w], src_v)
    for i in range(BLK):
        for j in range(D // 16):
            rows0[i, pl.ds(j * 16, 16)] = _zero_vec()

    def zero_step(i, carry):
        pltpu.sync_copy(rows0.at[pl.ds(0, BLK)],
                        acc.at[pl.ds(s * ROWS_T + i * BLK, BLK)])
        return carry

    lax.fori_loop(0, ROWS_T // BLK, zero_step, 0)
    plsc.subcore_barrier()

    pltpu.async_copy(u_hbm.at[src_v.at[0]], rows0, sem0)

    def block(k, carry):
        pltpu.sync_copy(dst32.at[w, pl.ds(k * DBLK, DBLK)], dstb)

        def pair(i, carry2):
            g = k * DBLK + 2 * i
            pltpu.async_copy(u_hbm.at[src_v.at[g + 1]], rows1, sem1)
            pltpu.make_async_copy(u_hbm.at[src_v.at[g]], rows0, sem0).wait()
            pltpu.sync_copy(rows0, acc.at[dstb.at[2 * i]], add=True)
            pltpu.async_copy(u_hbm.at[src_v.at[g + 2]], rows0, sem0)
            pltpu.make_async_copy(u_hbm.at[src_v.at[g + 1]], rows1, sem1).wait()
            pltpu.sync_copy(rows1, acc.at[dstb.at[2 * i + 1]], add=True)
            return carry2

        npairs = DBLK // 2
        lax.fori_loop(0, npairs, pair, 0)
        return carry

    lax.fori_loop(0, NBLK - 1, block, 0)
    k = NBLK - 1
    pltpu.sync_copy(dst32.at[w, pl.ds(k * DBLK, DBLK)], dstb)

    def pair_last(i, carry2):
        g = k * DBLK + 2 * i
        pltpu.async_copy(u_hbm.at[src_v.at[g + 1]], rows1, sem1)
        pltpu.make_async_copy(u_hbm.at[src_v.at[g]], rows0, sem0).wait()
        pltpu.sync_copy(rows0, acc.at[dstb.at[2 * i]], add=True)
        pltpu.async_copy(u_hbm.at[src_v.at[g + 2]], rows0, sem0)
        pltpu.make_async_copy(u_hbm.at[src_v.at[g + 1]], rows1, sem1).wait()
        pltpu.sync_copy(rows1, acc.at[dstb.at[2 * i + 1]], add=True)
        return carry2

    lax.fori_loop(0, DBLK // 2 - 1, pair_last, 0)
    g = NCH - 2
    pltpu.async_copy(u_hbm.at[src_v.at[g + 1]], rows1, sem1)
    pltpu.make_async_copy(u_hbm.at[src_v.at[g]], rows0, sem0).wait()
    pltpu.sync_copy(rows0, acc.at[dstb.at[DBLK - 2]], add=True)
    pltpu.make_async_copy(u_hbm.at[src_v.at[g + 1]], rows1, sem1).wait()
    pltpu.sync_copy(rows1, acc.at[dstb.at[DBLK - 1]], add=True)

    plsc.subcore_barrier()
    pltpu.sync_copy(acc.at[pl.ds(s * ROWS_T, ROWS_T)],
                    out_hbm.at[c, pl.ds(s * ROWS_T, ROWS_T)])


_agg_kernel = pl.kernel(
    _agg_body,
    out_type=jax.ShapeDtypeStruct((NC, NP, D), jnp.float32),
    mesh=_mesh,
    scratch_types=[
        pltpu.VMEM_SHARED((NP, D), jnp.float32),
        pltpu.VMEM((NCH, CH), jnp.int32),
        pltpu.VMEM((DBLK, CH), jnp.int32),
        pltpu.VMEM((CH, D), jnp.float32),
        pltpu.VMEM((CH, D), jnp.float32),
        pltpu.SemaphoreType.DMA,
        pltpu.SemaphoreType.DMA,
    ],
)


def _tc_first_body(deg_ref, x_ref, w_ref, dis_ref, u_ref):
    deg = deg_ref[0, :, 0:1] + deg_ref[1, :, 0:1] + 1.0
    dis = lax.rsqrt(deg)
    dis_ref[...] = dis
    h = jnp.dot(x_ref[...], w_ref[...], preferred_element_type=jnp.float32)
    u_ref[...] = dis * h


_tc_first = pl.pallas_call(
    _tc_first_body,
    out_shape=(
        jax.ShapeDtypeStruct((NP, 1), jnp.float32),
        jax.ShapeDtypeStruct((NP, D), jnp.float32),
    ),
)


def _tc_mid_body(sp_ref, u_ref, dis_ref, b_ref, w_ref, un_ref):
    dis = dis_ref[...]
    agg = sp_ref[0] + sp_ref[1] + u_ref[...]
    a = jnp.maximum(dis * agg + b_ref[...], 0.0)
    un_ref[...] = dis * jnp.dot(a, w_ref[...],
                                preferred_element_type=jnp.float32)


_tc_mid = pl.pallas_call(
    _tc_mid_body,
    out_shape=jax.ShapeDtypeStruct((NP, D), jnp.float32),
)


def _tc_last_body(sp_ref, u_ref, dis_ref, b_ref, out_ref):
    agg = sp_ref[0] + sp_ref[1] + u_ref[...]
    out_ref[...] = jnp.maximum(dis_ref[...] * agg + b_ref[...], 0.0)


_tc_last = pl.pallas_call(
    _tc_last_body,
    out_shape=jax.ShapeDtypeStruct((NP, D), jnp.float32),
)


def _pad_edges(idx):
    per_tile = idx.reshape(NW, E // NW)
    pad = N + (jnp.arange(E_T - E // NW, dtype=jnp.int32) % (NP - N))
    pad = jnp.broadcast_to(pad, (NW, E_T - E // NW))
    return jnp.concatenate([per_tile, pad], axis=1).reshape(NW, NCH, CH)


def kernel(x, edge_index, W1, b1, W2, b2, W3, b3):
    src32 = _pad_edges(edge_index[0].astype(jnp.int32))
    dst32 = _pad_edges(edge_index[1].astype(jnp.int32))
    xp = jnp.zeros((NP, D), jnp.float32).at[:N].set(x)
    b1 = b1.reshape(1, D)
    b2 = b2.reshape(1, D)
    b3 = b3.reshape(1, D)

    deg_p = _deg_kernel(dst32)
    dis, u1 = _tc_first(deg_p, xp, W1)
    s1 = _agg_kernel(u1, src32, dst32)
    u2 = _tc_mid(s1, u1, dis, b1, W2)
    s2 = _agg_kernel(u2, src32, dst32)
    u3 = _tc_mid(s2, u2, dis, b2, W3)
    s3 = _agg_kernel(u3, src32, dst32)
    out = _tc_last(s3, u3, dis, b3)
    return out[:N]


# speedup vs baseline: 25.7573x; 1.0324x over previous
import jax
import jax.numpy as jnp
from jax import lax
from jax.experimental import pallas as pl
from jax.experimental.pallas import tpu as pltpu
from jax.experimental.pallas import tpu_sc as plsc

N = 10000
NP = 10112
D = 128
E = 320000
NC = 2
NS = 16
NW = NC * NS
CH = 128
E_T = 10240
NCH = E_T // CH
BLK = 8
ROWS_T = NP // NS
ZB = 32
NZF = ROWS_T // ZB
ZTAIL = ROWS_T - NZF * ZB

_mesh = plsc.VectorSubcoreMesh(core_axis_name="c", subcore_axis_name="s",
                               num_cores=NC, num_subcores=NS)


def _zero_vec():
    return jnp.zeros((16,), jnp.float32)


def _zero_acc(acc, zbuf, semz, s):
    for i in range(ZB):
        for j in range(D // 16):
            zbuf[i, pl.ds(j * 16, 16)] = _zero_vec()
    base = s * ROWS_T

    def fire(i, carry):
        pltpu.async_copy(zbuf, acc.at[pl.ds(base + i * ZB, ZB)], semz)
        return carry

    lax.fori_loop(0, NZF, fire, 0)
    pltpu.async_copy(zbuf.at[pl.ds(0, ZTAIL)],
                     acc.at[pl.ds(base + NZF * ZB, ZTAIL)], semz)

    def drain(i, carry):
        pltpu.make_async_copy(zbuf, acc.at[pl.ds(base, ZB)], semz).wait()
        return carry

    lax.fori_loop(0, NZF, drain, 0)
    pltpu.make_async_copy(zbuf.at[pl.ds(0, ZTAIL)],
                          acc.at[pl.ds(base, ZTAIL)], semz).wait()


def _deg_body(dst32, deg_out, acc, dst_v, ones_v, zbuf, semz):
    c = lax.axis_index("c")
    s = lax.axis_index("s")
    w = c * NS + s
    pltpu.sync_copy(dst32.at[w], dst_v)
    for i in range(CH):
        for j in range(D // 16):
            ones_v[i, pl.ds(j * 16, 16)] = jnp.ones((16,), jnp.float32)
    _zero_acc(acc, zbuf, semz, s)
    plsc.subcore_barrier()

    def chunk(g, carry):
        pltpu.sync_copy(ones_v, acc.at[dst_v.at[g]], add=True)
        return carry

    lax.fori_loop(0, NCH, chunk, 0)
    plsc.subcore_barrier()
    pltpu.sync_copy(acc.at[pl.ds(s * ROWS_T, ROWS_T)],
                    deg_out.at[c, pl.ds(s * ROWS_T, ROWS_T)])


_deg_kernel = pl.kernel(
    _deg_body,
    out_type=jax.ShapeDtypeStruct((NC, NP, D), jnp.float32),
    mesh=_mesh,
    scratch_types=[
        pltpu.VMEM_SHARED((NP, D), jnp.float32),
        pltpu.VMEM((NCH, CH), jnp.int32),
        pltpu.VMEM((CH, D), jnp.float32),
        pltpu.VMEM((ZB, D), jnp.float32),
        pltpu.SemaphoreType.DMA,
    ],
)


DBLK = 16
NBLK = NCH // DBLK


def _agg_body(u_hbm, src32, dst32, out_hbm, acc, src_v, dstb,
              rows0, rows1, zbuf, sem0, sem1):
    c = lax.axis_index("c")
    s = lax.axis_index("s")
    w = c * NS + s
    pltpu.sync_copy(src32.at[w], src_v)
    _zero_acc(acc, zbuf, sem0, s)
    plsc.subcore_barrier()

    pltpu.async_copy(u_hbm.at[src_v.at[0]], rows0, sem0)

    def block(k, carry):
        pltpu.sync_copy(dst32.at[w, pl.ds(k * DBLK, DBLK)], dstb)

        def pair(i, carry2):
            g = k * DBLK + 2 * i
            pltpu.async_copy(u_hbm.at[src_v.at[g + 1]], rows1, sem1)
            pltpu.make_async_copy(u_hbm.at[src_v.at[g]], rows0, sem0).wait()
            pltpu.sync_copy(rows0, acc.at[dstb.at[2 * i]], add=True)
            pltpu.async_copy(u_hbm.at[src_v.at[g + 2]], rows0, sem0)
            pltpu.make_async_copy(u_hbm.at[src_v.at[g + 1]], rows1, sem1).wait()
            pltpu.sync_copy(rows1, acc.at[dstb.at[2 * i + 1]], add=True)
            return carry2

        npairs = DBLK // 2
        lax.fori_loop(0, npairs, pair, 0)
        return carry

    lax.fori_loop(0, NBLK - 1, block, 0)
    k = NBLK - 1
    pltpu.sync_copy(dst32.at[w, pl.ds(k * DBLK, DBLK)], dstb)

    def pair_last(i, carry2):
        g = k * DBLK + 2 * i
        pltpu.async_copy(u_hbm.at[src_v.at[g + 1]], rows1, sem1)
        pltpu.make_async_copy(u_hbm.at[src_v.at[g]], rows0, sem0).wait()
        pltpu.sync_copy(rows0, acc.at[dstb.at[2 * i]], add=True)
        pltpu.async_copy(u_hbm.at[src_v.at[g + 2]], rows0, sem0)
        pltpu.make_async_copy(u_hbm.at[src_v.at[g + 1]], rows1, sem1).wait()
        pltpu.sync_copy(rows1, acc.at[dstb.at[2 * i + 1]], add=True)
        return carry2

    lax.fori_loop(0, DBLK // 2 - 1, pair_last, 0)
    g = NCH - 2
    pltpu.async_copy(u_hbm.at[src_v.at[g + 1]], rows1, sem1)
    pltpu.make_async_copy(u_hbm.at[src_v.at[g]], rows0, sem0).wait()
    pltpu.sync_copy(rows0, acc.at[dstb.at[DBLK - 2]], add=True)
    pltpu.make_async_copy(u_hbm.at[src_v.at[g + 1]], rows1, sem1).wait()
    pltpu.sync_copy(rows1, acc.at[dstb.at[DBLK - 1]], add=True)

    plsc.subcore_barrier()
    pltpu.sync_copy(acc.at[pl.ds(s * ROWS_T, ROWS_T)],
                    out_hbm.at[c, pl.ds(s * ROWS_T, ROWS_T)])


_agg_kernel = pl.kernel(
    _agg_body,
    out_type=jax.ShapeDtypeStruct((NC, NP, D), jnp.float32),
    mesh=_mesh,
    scratch_types=[
        pltpu.VMEM_SHARED((NP, D), jnp.float32),
        pltpu.VMEM((NCH, CH), jnp.int32),
        pltpu.VMEM((DBLK, CH), jnp.int32),
        pltpu.VMEM((CH, D), jnp.float32),
        pltpu.VMEM((CH, D), jnp.float32),
        pltpu.VMEM((ZB, D), jnp.float32),
        pltpu.SemaphoreType.DMA,
        pltpu.SemaphoreType.DMA,
    ],
)


def _tc_first_body(deg_ref, x_ref, w_ref, dis_ref, u_ref):
    deg = deg_ref[0, :, 0:1] + deg_ref[1, :, 0:1] + 1.0
    dis = lax.rsqrt(deg)
    dis_ref[...] = dis
    h = jnp.dot(x_ref[...], w_ref[...], preferred_element_type=jnp.float32)
    u_ref[...] = dis * h


_tc_first = pl.pallas_call(
    _tc_first_body,
    out_shape=(
        jax.ShapeDtypeStruct((NP, 1), jnp.float32),
        jax.ShapeDtypeStruct((NP, D), jnp.float32),
    ),
)


def _tc_mid_body(sp_ref, u_ref, dis_ref, b_ref, w_ref, un_ref):
    dis = dis_ref[...]
    agg = sp_ref[0] + sp_ref[1] + u_ref[...]
    a = jnp.maximum(dis * agg + b_ref[...], 0.0)
    un_ref[...] = dis * jnp.dot(a, w_ref[...],
                                preferred_element_type=jnp.float32)


_tc_mid = pl.pallas_call(
    _tc_mid_body,
    out_shape=jax.ShapeDtypeStruct((NP, D), jnp.float32),
)


def _tc_last_body(sp_ref, u_ref, dis_ref, b_ref, out_ref):
    agg = sp_ref[0] + sp_ref[1] + u_ref[...]
    out_ref[...] = jnp.maximum(dis_ref[...] * agg + b_ref[...], 0.0)


_tc_last = pl.pallas_call(
    _tc_last_body,
    out_shape=jax.ShapeDtypeStruct((NP, D), jnp.float32),
)


def _pad_edges(idx):
    per_tile = idx.reshape(NW, E // NW)
    pad = N + (jnp.arange(E_T - E // NW, dtype=jnp.int32) % (NP - N))
    pad = jnp.broadcast_to(pad, (NW, E_T - E // NW))
    return jnp.concatenate([per_tile, pad], axis=1).reshape(NW, NCH, CH)


def kernel(x, edge_index, W1, b1, W2, b2, W3, b3):
    src32 = _pad_edges(edge_index[0].astype(jnp.int32))
    dst32 = _pad_edges(edge_index[1].astype(jnp.int32))
    xp = jnp.zeros((NP, D), jnp.float32).at[:N].set(x)
    b1 = b1.reshape(1, D)
    b2 = b2.reshape(1, D)
    b3 = b3.reshape(1, D)

    deg_p = _deg_kernel(dst32)
    dis, u1 = _tc_first(deg_p, xp, W1)
    s1 = _agg_kernel(u1, src32, dst32)
    u2 = _tc_mid(s1, u1, dis, b1, W2)
    s2 = _agg_kernel(u2, src32, dst32)
    u3 = _tc_mid(s2, u2, dis, b2, W3)
    s3 = _agg_kernel(u3, src32, dst32)
    out = _tc_last(s3, u3, dis, b3)
    return out[:N]


# speedup vs baseline: 26.0087x; 1.0098x over previous
import jax
import jax.numpy as jnp
from jax import lax
from jax.experimental import pallas as pl
from jax.experimental.pallas import tpu as pltpu
from jax.experimental.pallas import tpu_sc as plsc

N = 10000
NP = 10112
D = 128
E = 320000
NC = 2
NS = 16
NW = NC * NS
CH = 128
E_T = 10240
NCH = E_T // CH
BLK = 8
ROWS_T = NP // NS
ZB = 32
NZF = ROWS_T // ZB
ZTAIL = ROWS_T - NZF * ZB

_mesh = plsc.VectorSubcoreMesh(core_axis_name="c", subcore_axis_name="s",
                               num_cores=NC, num_subcores=NS)


def _zero_vec():
    return jnp.zeros((16,), jnp.float32)


def _zero_acc(acc, zbuf, semz, s):
    for i in range(ZB):
        for j in range(D // 16):
            zbuf[i, pl.ds(j * 16, 16)] = _zero_vec()
    base = s * ROWS_T

    def fire(i, carry):
        pltpu.async_copy(zbuf, acc.at[pl.ds(base + i * ZB, ZB)], semz)
        return carry

    lax.fori_loop(0, NZF, fire, 0)
    pltpu.async_copy(zbuf.at[pl.ds(0, ZTAIL)],
                     acc.at[pl.ds(base + NZF * ZB, ZTAIL)], semz)

    def drain(i, carry):
        pltpu.make_async_copy(zbuf, acc.at[pl.ds(base, ZB)], semz).wait()
        return carry

    lax.fori_loop(0, NZF, drain, 0)
    pltpu.make_async_copy(zbuf.at[pl.ds(0, ZTAIL)],
                          acc.at[pl.ds(base, ZTAIL)], semz).wait()


def _deg_body(dst32, deg_out, acc, dst_v, ones_v, zbuf, semz):
    c = lax.axis_index("c")
    s = lax.axis_index("s")
    w = c * NS + s
    pltpu.sync_copy(dst32.at[w], dst_v)
    for i in range(CH):
        for j in range(D // 16):
            ones_v[i, pl.ds(j * 16, 16)] = jnp.ones((16,), jnp.float32)
    _zero_acc(acc, zbuf, semz, s)
    plsc.subcore_barrier()

    def chunk(g, carry):
        pltpu.sync_copy(ones_v, acc.at[dst_v.at[g]], add=True)
        return carry

    lax.fori_loop(0, NCH, chunk, 0)
    plsc.subcore_barrier()
    pltpu.sync_copy(acc.at[pl.ds(s * ROWS_T, ROWS_T)],
                    deg_out.at[c, pl.ds(s * ROWS_T, ROWS_T)])


_deg_kernel = pl.kernel(
    _deg_body,
    out_type=jax.ShapeDtypeStruct((NC, NP, D), jnp.float32),
    mesh=_mesh,
    scratch_types=[
        pltpu.VMEM_SHARED((NP, D), jnp.float32),
        pltpu.VMEM((NCH, CH), jnp.int32),
        pltpu.VMEM((CH, D), jnp.float32),
        pltpu.VMEM((ZB, D), jnp.float32),
        pltpu.SemaphoreType.DMA,
    ],
)


DBLK = 16
NBLK = NCH // DBLK


def _agg_body(u_hbm, src32, dst32, out_hbm, acc, src_v, dstb,
              rows0, rows1, zbuf, sem0, sem1):
    c = lax.axis_index("c")
    s = lax.axis_index("s")
    w = c * NS + s
    pltpu.sync_copy(src32.at[w], src_v)
    _zero_acc(acc, zbuf, sem0, s)
    plsc.subcore_barrier()

    pltpu.async_copy(u_hbm.at[src_v.at[0]], rows0, sem0)

    def block(k, carry):
        pltpu.sync_copy(dst32.at[w, pl.ds(k * DBLK, DBLK)], dstb)

        def pair(i, carry2):
            g = k * DBLK + 2 * i
            pltpu.async_copy(u_hbm.at[src_v.at[g + 1]], rows1, sem1)
            pltpu.make_async_copy(u_hbm.at[src_v.at[g]], rows0, sem0).wait()
            pltpu.sync_copy(rows0, acc.at[dstb.at[2 * i]], add=True)
            pltpu.async_copy(u_hbm.at[src_v.at[g + 2]], rows0, sem0)
            pltpu.make_async_copy(u_hbm.at[src_v.at[g + 1]], rows1, sem1).wait()
            pltpu.sync_copy(rows1, acc.at[dstb.at[2 * i + 1]], add=True)
            return carry2

        npairs = DBLK // 2
        lax.fori_loop(0, npairs, pair, 0)
        return carry

    lax.fori_loop(0, NBLK - 1, block, 0)
    k = NBLK - 1
    pltpu.sync_copy(dst32.at[w, pl.ds(k * DBLK, DBLK)], dstb)

    def pair_last(i, carry2):
        g = k * DBLK + 2 * i
        pltpu.async_copy(u_hbm.at[src_v.at[g + 1]], rows1, sem1)
        pltpu.make_async_copy(u_hbm.at[src_v.at[g]], rows0, sem0).wait()
        pltpu.sync_copy(rows0, acc.at[dstb.at[2 * i]], add=True)
        pltpu.async_copy(u_hbm.at[src_v.at[g + 2]], rows0, sem0)
        pltpu.make_async_copy(u_hbm.at[src_v.at[g + 1]], rows1, sem1).wait()
        pltpu.sync_copy(rows1, acc.at[dstb.at[2 * i + 1]], add=True)
        return carry2

    lax.fori_loop(0, DBLK // 2 - 1, pair_last, 0)
    g = NCH - 2
    pltpu.async_copy(u_hbm.at[src_v.at[g + 1]], rows1, sem1)
    pltpu.make_async_copy(u_hbm.at[src_v.at[g]], rows0, sem0).wait()
    pltpu.sync_copy(rows0, acc.at[dstb.at[DBLK - 2]], add=True)
    pltpu.make_async_copy(u_hbm.at[src_v.at[g + 1]], rows1, sem1).wait()
    pltpu.sync_copy(rows1, acc.at[dstb.at[DBLK - 1]], add=True)

    plsc.subcore_barrier()
    pltpu.sync_copy(acc.at[pl.ds(s * ROWS_T, ROWS_T)],
                    out_hbm.at[c, pl.ds(s * ROWS_T, ROWS_T)])


_agg_kernel = pl.kernel(
    _agg_body,
    out_type=jax.ShapeDtypeStruct((NC, NP, D), jnp.float32),
    mesh=_mesh,
    scratch_types=[
        pltpu.VMEM_SHARED((NP, D), jnp.float32),
        pltpu.VMEM((NCH, CH), jnp.int32),
        pltpu.VMEM((DBLK, CH), jnp.int32),
        pltpu.VMEM((CH, D), jnp.float32),
        pltpu.VMEM((CH, D), jnp.float32),
        pltpu.VMEM((ZB, D), jnp.float32),
        pltpu.SemaphoreType.DMA,
        pltpu.SemaphoreType.DMA,
    ],
)


def _tc_matmul_body(x_ref, w_ref, h_ref):
    h_ref[pl.ds(0, N), :] = jnp.dot(x_ref[...], w_ref[...],
                                    preferred_element_type=jnp.float32)
    h_ref[pl.ds(N, NP - N), :] = jnp.zeros((NP - N, D), jnp.float32)


_tc_matmul = pl.pallas_call(
    _tc_matmul_body,
    out_shape=jax.ShapeDtypeStruct((NP, D), jnp.float32),
)


def _tc_udis_body(deg_ref, h_ref, dis_ref, u_ref):
    deg = deg_ref[0, :, 0:1] + deg_ref[1, :, 0:1] + 1.0
    dis = lax.rsqrt(deg)
    dis_ref[...] = dis
    u_ref[...] = dis * h_ref[...]


_tc_udis = pl.pallas_call(
    _tc_udis_body,
    out_shape=(
        jax.ShapeDtypeStruct((NP, 1), jnp.float32),
        jax.ShapeDtypeStruct((NP, D), jnp.float32),
    ),
)


def _tc_mid_body(sp_ref, u_ref, dis_ref, b_ref, w_ref, un_ref):
    dis = dis_ref[...]
    agg = sp_ref[0] + sp_ref[1] + u_ref[...]
    a = jnp.maximum(dis * agg + b_ref[...], 0.0)
    un_ref[...] = dis * jnp.dot(a, w_ref[...],
                                preferred_element_type=jnp.float32)


_tc_mid = pl.pallas_call(
    _tc_mid_body,
    out_shape=jax.ShapeDtypeStruct((NP, D), jnp.float32),
)


def _tc_last_body(sp_ref, u_ref, dis_ref, b_ref, out_ref):
    agg = (sp_ref[0, pl.ds(0, N), :] + sp_ref[1, pl.ds(0, N), :]
           + u_ref[pl.ds(0, N), :])
    out_ref[...] = jnp.maximum(
        dis_ref[pl.ds(0, N), :] * agg + b_ref[...], 0.0)


_tc_last = pl.pallas_call(
    _tc_last_body,
    out_shape=jax.ShapeDtypeStruct((N, D), jnp.float32),
)


def _pad_edges(idx):
    per_tile = idx.reshape(NW, E // NW)
    pad = N + (jnp.arange(E_T - E // NW, dtype=jnp.int32) % (NP - N))
    pad = jnp.broadcast_to(pad, (NW, E_T - E // NW))
    return jnp.concatenate([per_tile, pad], axis=1).reshape(NW, NCH, CH)


def kernel(x, edge_index, W1, b1, W2, b2, W3, b3):
    src32 = _pad_edges(edge_index[0].astype(jnp.int32))
    dst32 = _pad_edges(edge_index[1].astype(jnp.int32))
    b1 = b1.reshape(1, D)
    b2 = b2.reshape(1, D)
    b3 = b3.reshape(1, D)

    h1 = _tc_matmul(x, W1)
    deg_p = _deg_kernel(dst32)
    dis, u1 = _tc_udis(deg_p, h1)
    s1 = _agg_kernel(u1, src32, dst32)
    u2 = _tc_mid(s1, u1, dis, b1, W2)
    s2 = _agg_kernel(u2, src32, dst32)
    u3 = _tc_mid(s2, u2, dis, b2, W3)
    s3 = _agg_kernel(u3, src32, dst32)
    return _tc_last(s3, u3, dis, b3)


# speedup vs baseline: 26.0859x; 1.0030x over previous
import jax
import jax.numpy as jnp
from jax import lax
from jax.experimental import pallas as pl
from jax.experimental.pallas import tpu as pltpu
from jax.experimental.pallas import tpu_sc as plsc

N = 10000
NP = 10112
D = 128
E = 320000
NC = 2
NS = 16
NW = NC * NS
CH = 128
E_T = 10240
NCH = E_T // CH
BLK = 8
ROWS_T = NP // NS
ZB = 32
NZF = ROWS_T // ZB
ZTAIL = ROWS_T - NZF * ZB

_mesh = plsc.VectorSubcoreMesh(core_axis_name="c", subcore_axis_name="s",
                               num_cores=NC, num_subcores=NS)


def _zero_vec():
    return jnp.zeros((16,), jnp.float32)


def _zero_acc(acc, zbuf, semz, s):
    for i in range(ZB):
        for j in range(D // 16):
            zbuf[i, pl.ds(j * 16, 16)] = _zero_vec()
    base = s * ROWS_T

    def fire(i, carry):
        pltpu.async_copy(zbuf, acc.at[pl.ds(base + i * ZB, ZB)], semz)
        return carry

    lax.fori_loop(0, NZF, fire, 0)
    pltpu.async_copy(zbuf.at[pl.ds(0, ZTAIL)],
                     acc.at[pl.ds(base + NZF * ZB, ZTAIL)], semz)

    def drain(i, carry):
        pltpu.make_async_copy(zbuf, acc.at[pl.ds(base, ZB)], semz).wait()
        return carry

    lax.fori_loop(0, NZF, drain, 0)
    pltpu.make_async_copy(zbuf.at[pl.ds(0, ZTAIL)],
                          acc.at[pl.ds(base, ZTAIL)], semz).wait()


def _deg_body(dst32, deg_out, acc, dst_v, ones_v, zbuf, semz):
    c = lax.axis_index("c")
    s = lax.axis_index("s")
    w = c * NS + s
    pltpu.sync_copy(dst32.at[w], dst_v)
    for i in range(CH):
        for j in range(D // 16):
            ones_v[i, pl.ds(j * 16, 16)] = jnp.ones((16,), jnp.float32)
    _zero_acc(acc, zbuf, semz, s)
    plsc.subcore_barrier()

    for b in range(BLK):
        pltpu.async_copy(ones_v, acc.at[dst_v.at[b]], semz, add=True)

    def chunk(k, carry):
        for b in range(BLK):
            pltpu.async_copy(ones_v, acc.at[dst_v.at[k * BLK + b]],
                             semz, add=True)
        for b in range(BLK):
            pltpu.make_async_copy(ones_v, acc.at[dst_v.at[0]], semz).wait()
        return carry

    lax.fori_loop(1, NCH // BLK, chunk, 0)
    for b in range(BLK):
        pltpu.make_async_copy(ones_v, acc.at[dst_v.at[0]], semz).wait()
    plsc.subcore_barrier()
    pltpu.sync_copy(acc.at[pl.ds(s * ROWS_T, ROWS_T)],
                    deg_out.at[c, pl.ds(s * ROWS_T, ROWS_T)])


_deg_kernel = pl.kernel(
    _deg_body,
    out_type=jax.ShapeDtypeStruct((NC, NP, D), jnp.float32),
    mesh=_mesh,
    scratch_types=[
        pltpu.VMEM_SHARED((NP, D), jnp.float32),
        pltpu.VMEM((NCH, CH), jnp.int32),
        pltpu.VMEM((CH, D), jnp.float32),
        pltpu.VMEM((ZB, D), jnp.float32),
        pltpu.SemaphoreType.DMA,
    ],
)


DBLK = 16
NBLK = NCH // DBLK


def _agg_body(u_hbm, src32, dst32, out_hbm, acc, src_v, dstb,
              rows0, rows1, zbuf, sem0, sem1):
    c = lax.axis_index("c")
    s = lax.axis_index("s")
    w = c * NS + s
    pltpu.sync_copy(src32.at[w], src_v)
    _zero_acc(acc, zbuf, sem0, s)
    plsc.subcore_barrier()

    pltpu.async_copy(u_hbm.at[src_v.at[0]], rows0, sem0)

    def block(k, carry):
        pltpu.sync_copy(dst32.at[w, pl.ds(k * DBLK, DBLK)], dstb)

        def pair(i, carry2):
            g = k * DBLK + 2 * i
            pltpu.async_copy(u_hbm.at[src_v.at[g + 1]], rows1, sem1)
            pltpu.make_async_copy(u_hbm.at[src_v.at[g]], rows0, sem0).wait()
            pltpu.sync_copy(rows0, acc.at[dstb.at[2 * i]], add=True)
            pltpu.async_copy(u_hbm.at[src_v.at[g + 2]], rows0, sem0)
            pltpu.make_async_copy(u_hbm.at[src_v.at[g + 1]], rows1, sem1).wait()
            pltpu.sync_copy(rows1, acc.at[dstb.at[2 * i + 1]], add=True)
            return carry2

        npairs = DBLK // 2
        lax.fori_loop(0, npairs, pair, 0)
        return carry

    lax.fori_loop(0, NBLK - 1, block, 0)
    k = NBLK - 1
    pltpu.sync_copy(dst32.at[w, pl.ds(k * DBLK, DBLK)], dstb)

    def pair_last(i, carry2):
        g = k * DBLK + 2 * i
        pltpu.async_copy(u_hbm.at[src_v.at[g + 1]], rows1, sem1)
        pltpu.make_async_copy(u_hbm.at[src_v.at[g]], rows0, sem0).wait()
        pltpu.sync_copy(rows0, acc.at[dstb.at[2 * i]], add=True)
        pltpu.async_copy(u_hbm.at[src_v.at[g + 2]], rows0, sem0)
        pltpu.make_async_copy(u_hbm.at[src_v.at[g + 1]], rows1, sem1).wait()
        pltpu.sync_copy(rows1, acc.at[dstb.at[2 * i + 1]], add=True)
        return carry2

    lax.fori_loop(0, DBLK // 2 - 1, pair_last, 0)
    g = NCH - 2
    pltpu.async_copy(u_hbm.at[src_v.at[g + 1]], rows1, sem1)
    pltpu.make_async_copy(u_hbm.at[src_v.at[g]], rows0, sem0).wait()
    pltpu.sync_copy(rows0, acc.at[dstb.at[DBLK - 2]], add=True)
    pltpu.make_async_copy(u_hbm.at[src_v.at[g + 1]], rows1, sem1).wait()
    pltpu.sync_copy(rows1, acc.at[dstb.at[DBLK - 1]], add=True)

    plsc.subcore_barrier()
    pltpu.sync_copy(acc.at[pl.ds(s * ROWS_T, ROWS_T)],
                    out_hbm.at[c, pl.ds(s * ROWS_T, ROWS_T)])


_agg_kernel = pl.kernel(
    _agg_body,
    out_type=jax.ShapeDtypeStruct((NC, NP, D), jnp.float32),
    mesh=_mesh,
    scratch_types=[
        pltpu.VMEM_SHARED((NP, D), jnp.float32),
        pltpu.VMEM((NCH, CH), jnp.int32),
        pltpu.VMEM((DBLK, CH), jnp.int32),
        pltpu.VMEM((CH, D), jnp.float32),
        pltpu.VMEM((CH, D), jnp.float32),
        pltpu.VMEM((ZB, D), jnp.float32),
        pltpu.SemaphoreType.DMA,
        pltpu.SemaphoreType.DMA,
    ],
)


def _tc_matmul_body(x_ref, w_ref, h_ref):
    h_ref[pl.ds(0, N), :] = jnp.dot(x_ref[...], w_ref[...],
                                    preferred_element_type=jnp.float32)
    h_ref[pl.ds(N, NP - N), :] = jnp.zeros((NP - N, D), jnp.float32)


_tc_matmul = pl.pallas_call(
    _tc_matmul_body,
    out_shape=jax.ShapeDtypeStruct((NP, D), jnp.float32),
)


def _tc_udis_body(deg_ref, h_ref, dis_ref, u_ref):
    deg = deg_ref[0, :, 0:1] + deg_ref[1, :, 0:1] + 1.0
    dis = lax.rsqrt(deg)
    dis_ref[...] = dis
    u_ref[...] = dis * h_ref[...]


_tc_udis = pl.pallas_call(
    _tc_udis_body,
    out_shape=(
        jax.ShapeDtypeStruct((NP, 1), jnp.float32),
        jax.ShapeDtypeStruct((NP, D), jnp.float32),
    ),
)


def _tc_mid_body(sp_ref, u_ref, dis_ref, b_ref, w_ref, un_ref):
    dis = dis_ref[...]
    agg = sp_ref[0] + sp_ref[1] + u_ref[...]
    a = jnp.maximum(dis * agg + b_ref[...], 0.0)
    un_ref[...] = dis * jnp.dot(a, w_ref[...],
                                preferred_element_type=jnp.float32)


_tc_mid = pl.pallas_call(
    _tc_mid_body,
    out_shape=jax.ShapeDtypeStruct((NP, D), jnp.float32),
)


def _tc_last_body(sp_ref, u_ref, dis_ref, b_ref, out_ref):
    agg = (sp_ref[0, pl.ds(0, N), :] + sp_ref[1, pl.ds(0, N), :]
           + u_ref[pl.ds(0, N), :])
    out_ref[...] = jnp.maximum(
        dis_ref[pl.ds(0, N), :] * agg + b_ref[...], 0.0)


_tc_last = pl.pallas_call(
    _tc_last_body,
    out_shape=jax.ShapeDtypeStruct((N, D), jnp.float32),
)


def _pad_edges(idx):
    per_tile = idx.reshape(NW, E // NW)
    pad = N + (jnp.arange(E_T - E // NW, dtype=jnp.int32) % (NP - N))
    pad = jnp.broadcast_to(pad, (NW, E_T - E // NW))
    return jnp.concatenate([per_tile, pad], axis=1).reshape(NW, NCH, CH)


def kernel(x, edge_index, W1, b1, W2, b2, W3, b3):
    src32 = _pad_edges(edge_index[0].astype(jnp.int32))
    dst32 = _pad_edges(edge_index[1].astype(jnp.int32))
    b1 = b1.reshape(1, D)
    b2 = b2.reshape(1, D)
    b3 = b3.reshape(1, D)

    h1 = _tc_matmul(x, W1)
    deg_p = _deg_kernel(dst32)
    dis, u1 = _tc_udis(deg_p, h1)
    s1 = _agg_kernel(u1, src32, dst32)
    u2 = _tc_mid(s1, u1, dis, b1, W2)
    s2 = _agg_kernel(u2, src32, dst32)
    u3 = _tc_mid(s2, u2, dis, b2, W3)
    s3 = _agg_kernel(u3, src32, dst32)
    return _tc_last(s3, u3, dis, b3)
